# baseline (speedup 1.0000x reference)
import functools

import jax
import jax.numpy as jnp
from jax import lax
from jax.experimental import pallas as pl
from jax.experimental.pallas import tpu as pltpu
from jax.experimental.pallas import tpu_sc as plsc

_NC = 2
_NS = 16
_CH = 80


def _sc_mesh():
    return plsc.VectorSubcoreMesh(
        core_axis_name="c", subcore_axis_name="s", num_cores=_NC, num_subcores=_NS
    )


def _wait_gather(tab_hbm, rows_ref, sem):
    pltpu.make_async_copy(tab_hbm.at[pl.ds(0, _CH)], rows_ref, sem).wait()


def _wait_scatter(rows_ref, acc_sh, sem):
    pltpu.make_async_copy(rows_ref, acc_sh.at[pl.ds(0, _CH)], sem).wait()


def _make_deg_kernel(NP, nch):
    rows_pt = NP // _NS
    epcp = nch * _CH
    EP = epcp * _NS
    ngrp = nch // 8

    @functools.partial(
        pl.kernel,
        mesh=_sc_mesh(),
        out_type=jax.ShapeDtypeStruct((2 * NP, 16), jnp.float32),
        scratch_types=(
            [pltpu.VMEM((_CH, 16), jnp.float32)]
            + [pltpu.VMEM((_CH,), jnp.int32) for _ in range(16)]
            + [
                pltpu.VMEM_SHARED((NP, 16), jnp.float32),
                pltpu.SemaphoreType.DMA,
                pltpu.SemaphoreType.DMA,
            ]
        ),
    )
    def deg_kernel(dst_hbm, ones_hbm, init_hbm, out_hbm, *scr):
        ones_v = scr[0]
        didx = [scr[1 + i] for i in range(16)]
        acc_sh = scr[17]
        isem = scr[18]
        zsem = scr[19]
        cid = lax.axis_index("c")
        sid = lax.axis_index("s")
        r0 = sid * rows_pt
        ebase = cid * EP + sid * epcp
        zcopy = pltpu.async_copy(
            init_hbm.at[pl.ds(r0, rows_pt)], acc_sh.at[pl.ds(r0, rows_pt)],
            zsem,
        )
        pltpu.sync_copy(ones_hbm, ones_v)
        for j in range(8):
            pltpu.async_copy(
                dst_hbm.at[pl.ds(ebase + j * _CH, _CH)], didx[j], isem
            )
        for j in range(8):
            pltpu.make_async_copy(
                dst_hbm.at[pl.ds(ebase + j * _CH, _CH)], didx[j], isem
            ).wait()
        zcopy.wait()
        plsc.subcore_barrier()

        def grp_pair(gp, carry):
            for half in range(2):
                g = 2 * gp + half
                other = 1 - half
                for j in range(8):
                    if j == 4:
                        @pl.when(g < ngrp - 1)
                        def _(g=g, other=other):
                            for jj in range(8):
                                nc = (g + 1) * 8 + jj
                                pltpu.async_copy(
                                    dst_hbm.at[pl.ds(ebase + nc * _CH, _CH)],
                                    didx[other * 8 + jj], isem,
                                )
                    if j == 6:
                        @pl.when(g < ngrp - 1)
                        def _(g=g, other=other):
                            for jj in range(8):
                                nc = (g + 1) * 8 + jj
                                pltpu.make_async_copy(
                                    dst_hbm.at[pl.ds(ebase + nc * _CH, _CH)],
                                    didx[other * 8 + jj], isem,
                                ).wait()
                    pltpu.sync_copy(
                        ones_v, acc_sh.at[didx[half * 8 + j]], add=True
                    )
            return carry

        lax.fori_loop(0, ngrp // 2, grp_pair, 0)
        plsc.subcore_barrier()
        pltpu.sync_copy(
            acc_sh.at[pl.ds(r0, rows_pt)],
            out_hbm.at[pl.ds(cid * NP + r0, rows_pt)],
        )

    return deg_kernel


def _make_agg_kernel(NP, nch, D):
    rows_pt = NP // _NS
    epcp = nch * _CH
    EP = epcp * _NS
    ngrp = nch // 8

    @functools.partial(
        pl.kernel,
        mesh=_sc_mesh(),
        out_type=jax.ShapeDtypeStruct((2 * NP, D), jnp.float32),
        scratch_types=(
            [pltpu.VMEM((_CH, D), jnp.float32) for _ in range(2)]
            + [pltpu.VMEM((_CH,), jnp.int32) for _ in range(32)]
            + [pltpu.VMEM_SHARED((NP, D), jnp.float32)]
            + [pltpu.SemaphoreType.DMA for _ in range(4)]
        ),
    )
    def agg_kernel(tab_hbm, src_hbm, dst_hbm, zeros_hbm, out_hbm, *scr):
        rows = [scr[0], scr[1]]
        sidx = [scr[2 + i] for i in range(16)]
        didx = [scr[18 + i] for i in range(16)]
        acc_sh = scr[34]
        gsem = [scr[35], scr[36]]
        isem = scr[37]
        zsem = scr[38]
        cid = lax.axis_index("c")
        sid = lax.axis_index("s")
        rr = sid * rows_pt
        ebase = cid * EP + sid * epcp
        zcopy = pltpu.async_copy(
            zeros_hbm.at[pl.ds(rr, rows_pt)], acc_sh.at[pl.ds(rr, rows_pt)],
            zsem,
        )
        for j in range(8):
            pltpu.async_copy(
                src_hbm.at[pl.ds(ebase + j * _CH, _CH)], sidx[j], isem
            )
            pltpu.async_copy(
                dst_hbm.at[pl.ds(ebase + j * _CH, _CH)], didx[j], isem
            )
        for j in range(8):
            pltpu.make_async_copy(
                src_hbm.at[pl.ds(ebase + j * _CH, _CH)], sidx[j], isem
            ).wait()
            pltpu.make_async_copy(
                dst_hbm.at[pl.ds(ebase + j * _CH, _CH)], didx[j], isem
            ).wait()
        pltpu.async_copy(tab_hbm.at[sidx[0]], rows[0], gsem[0])
        zcopy.wait()
        plsc.subcore_barrier()

        def grp_pair(gp, carry):
            handles = {}
            for half in range(2):
                g = 2 * gp + half
                other = 1 - half
                for j in range(8):
                    b = j % 2
                    nb = 1 - b
                    if (half, j) == (0, 0):
                        pltpu.make_async_copy(
                            tab_hbm.at[sidx[0]], rows[0], gsem[0]
                        ).wait()
                    else:
                        handles.pop(b).wait()
                    if j < 7:
                        handles[nb] = pltpu.async_copy(
                            tab_hbm.at[sidx[half * 8 + j + 1]], rows[nb],
                            gsem[nb],
                        )
                    elif half == 0:
                        handles[nb] = pltpu.async_copy(
                            tab_hbm.at[sidx[8]], rows[nb], gsem[nb]
                        )
                    else:
                        @pl.when(g < ngrp - 1)
                        def _(nb=nb):
                            pltpu.async_copy(
                                tab_hbm.at[sidx[0]], rows[nb], gsem[nb]
                            )
                    if j == 4:
                        @pl.when(g < ngrp - 1)
                        def _(g=g, other=other):
                            for jj in range(8):
                                nc = (g + 1) * 8 + jj
                                pltpu.async_copy(
                                    src_hbm.at[pl.ds(ebase + nc * _CH, _CH)],
                                    sidx[other * 8 + jj], isem,
                                )
                                pltpu.async_copy(
                                    dst_hbm.at[pl.ds(ebase + nc * _CH, _CH)],
                                    didx[other * 8 + jj], isem,
                                )
                    if j == 6:
                        @pl.when(g < ngrp - 1)
                        def _(g=g, other=other):
                            for jj in range(8):
                                nc = (g + 1) * 8 + jj
                                pltpu.make_async_copy(
                                    src_hbm.at[pl.ds(ebase + nc * _CH, _CH)],
                                    sidx[other * 8 + jj], isem,
                                ).wait()
                                pltpu.make_async_copy(
                                    dst_hbm.at[pl.ds(ebase + nc * _CH, _CH)],
                                    didx[other * 8 + jj], isem,
                                ).wait()
                    pltpu.sync_copy(
                        rows[b], acc_sh.at[didx[half * 8 + j]], add=True
                    )
            return carry

        lax.fori_loop(0, ngrp // 2, grp_pair, 0)
        plsc.subcore_barrier()
        pltpu.sync_copy(
            acc_sh.at[pl.ds(rr, rows_pt)],
            out_hbm.at[pl.ds(cid * NP + rr, rows_pt)],
        )

    return agg_kernel


def _make_state_gather_kernel(B, D):
    bpw = B // (_NC * _NS)

    @functools.partial(
        pl.kernel,
        mesh=_sc_mesh(),
        out_type=jax.ShapeDtypeStruct((B, D), jnp.float32),
        scratch_types=[
            pltpu.VMEM((bpw,), jnp.int32),
            pltpu.VMEM((bpw, D), jnp.float32),
            pltpu.SemaphoreType.DMA,
        ],
    )
    def gather_kernel(tab_hbm, idx_hbm, out_hbm, idx_v, rows_v, sem):
        wid = lax.axis_index("s") * _NC + lax.axis_index("c")
        base = wid * bpw
        pltpu.sync_copy(idx_hbm.at[pl.ds(base, bpw)], idx_v)
        pltpu.async_copy(tab_hbm.at[idx_v], rows_v, sem).wait()
        pltpu.sync_copy(rows_v, out_hbm.at[pl.ds(base, bpw)])

    return gather_kernel


def _prep_body(deg_ref, x_ref, dinv_ref, y0_ref):
    dinv = lax.rsqrt(deg_ref[:, 0:1])
    dinv_ref[:, :] = dinv
    y0_ref[:, :] = x_ref[:, :] * dinv


def _mid_body(agg_ref, y0_ref, dinv_ref, w1_ref, b1_ref, w2_ref, u_ref):
    dinv = dinv_ref[:, :]
    ax = (agg_ref[:, :] + y0_ref[:, :]) * dinv
    h1 = jnp.maximum(
        jnp.dot(ax, w1_ref[:, :], preferred_element_type=jnp.float32) + b1_ref[:],
        0.0,
    )
    u_ref[:, :] = (
        jnp.dot(h1, w2_ref[:, :], preferred_element_type=jnp.float32)
        * dinv
    )


def _final_body(agg_ref, u_ref, dinv_ref, b2_ref, g_ref):
    dinv = dinv_ref[:, :]
    g_ref[:, :] = dinv * (agg_ref[:, :] + u_ref[:, :]) + b2_ref[:]


def _head_body(gx_ref, gy_ref, wh_ref, bh_ref, wp_ref, bp_ref, out_ref):
    z = gx_ref[:, :] * gy_ref[:, :]
    o = jnp.maximum(
        jnp.dot(z, wh_ref[:, :], preferred_element_type=jnp.float32) + bh_ref[:],
        0.0,
    )
    p = jnp.dot(o, wp_ref[:, :], preferred_element_type=jnp.float32) + bp_ref[:]
    m = jnp.max(p, axis=1, keepdims=True)
    e = jnp.exp(p - m)
    out_ref[:, :] = e / jnp.sum(e, axis=1, keepdims=True)


def kernel(x_a, x_b, W1, b1, W2, b2, Wh, bh, Wp, bp,
           edge_index_a, edge_index_b, states):
    N, D = x_a.shape
    E = edge_index_a.shape[1]
    S = states.shape[0]
    H = W1.shape[1]
    NP = ((N + 511) // 512) * 512
    TN = 2 * NP

    epc = E // _NS
    nch = ((-(-epc // _CH) + 15) // 16) * 16
    EP = nch * _CH * _NS
    padn = EP - E

    X = jnp.zeros((TN, D), jnp.float32)
    X = lax.dynamic_update_slice(X, x_a, (0, 0))
    X = lax.dynamic_update_slice(X, x_b, (NP, 0))
    fill_src = jnp.zeros((padn,), jnp.int32)
    fill_dst = N + (jnp.arange(padn, dtype=jnp.int32) % (NP - N))
    src_all = jnp.concatenate([edge_index_a[0], fill_src,
                               edge_index_b[0] + NP, fill_src])
    dst_all = jnp.concatenate([edge_index_a[1], fill_dst,
                               edge_index_b[1], fill_dst])
    zerosD = jnp.zeros((NP, D), jnp.float32)
    ones16 = jnp.ones((_CH, 16), jnp.float32)
    ones_init = jnp.ones((NP, 16), jnp.float32)
    sidx = jnp.concatenate([states[:, 0], NP + states[:, 1]])

    deg16 = _make_deg_kernel(NP, nch)(dst_all, ones16, ones_init)

    R = 1024
    grid = (TN // R,)
    dinv, Y0 = pl.pallas_call(
        _prep_body,
        grid=grid,
        in_specs=[
            pl.BlockSpec((R, 16), lambda i: (i, 0)),
            pl.BlockSpec((R, D), lambda i: (i, 0)),
        ],
        out_specs=[
            pl.BlockSpec((R, 1), lambda i: (i, 0)),
            pl.BlockSpec((R, D), lambda i: (i, 0)),
        ],
        out_shape=[
            jax.ShapeDtypeStruct((TN, 1), jnp.float32),
            jax.ShapeDtypeStruct((TN, D), jnp.float32),
        ],
    )(deg16, X)

    agg_kernel = _make_agg_kernel(NP, nch, D)

    agg1 = agg_kernel(Y0, src_all, dst_all, zerosD)

    U = pl.pallas_call(
        _mid_body,
        grid=grid,
        in_specs=[
            pl.BlockSpec((R, D), lambda i: (i, 0)),
            pl.BlockSpec((R, D), lambda i: (i, 0)),
            pl.BlockSpec((R, 1), lambda i: (i, 0)),
            pl.BlockSpec((D, H), lambda i: (0, 0)),
            pl.BlockSpec((H,), lambda i: (0,)),
            pl.BlockSpec((H, D), lambda i: (0, 0)),
        ],
        out_specs=pl.BlockSpec((R, D), lambda i: (i, 0)),
        out_shape=jax.ShapeDtypeStruct((TN, D), jnp.float32),
    )(agg1, Y0, dinv, W1, b1, W2)

    agg2 = agg_kernel(U, src_all, dst_all, zerosD)

    G = pl.pallas_call(
        _final_body,
        grid=grid,
        in_specs=[
            pl.BlockSpec((R, D), lambda i: (i, 0)),
            pl.BlockSpec((R, D), lambda i: (i, 0)),
            pl.BlockSpec((R, 1), lambda i: (i, 0)),
            pl.BlockSpec((D,), lambda i: (0,)),
        ],
        out_specs=pl.BlockSpec((R, D), lambda i: (i, 0)),
        out_shape=jax.ShapeDtypeStruct((TN, D), jnp.float32),
    )(agg2, U, dinv, b2)

    gxy = _make_state_gather_kernel(2 * S, D)(G, sidx)

    policy = pl.pallas_call(
        _head_body,
        out_shape=jax.ShapeDtypeStruct((S, 2), jnp.float32),
    )(gxy[:S], gxy[S:], Wh, bh, Wp, bp)
    return policy


# speedup vs baseline: 1.0375x; 1.0375x over previous
import functools

import jax
import jax.numpy as jnp
from jax import lax
from jax.experimental import pallas as pl
from jax.experimental.pallas import tpu as pltpu
from jax.experimental.pallas import tpu_sc as plsc

_NC = 2
_NS = 16
_CH = 128


def _sc_mesh():
    return plsc.VectorSubcoreMesh(
        core_axis_name="c", subcore_axis_name="s", num_cores=_NC, num_subcores=_NS
    )


def _wait_gather(tab_hbm, rows_ref, sem):
    pltpu.make_async_copy(tab_hbm.at[pl.ds(0, _CH)], rows_ref, sem).wait()


def _wait_scatter(rows_ref, acc_sh, sem):
    pltpu.make_async_copy(rows_ref, acc_sh.at[pl.ds(0, _CH)], sem).wait()


def _make_deg_kernel(NP, nch):
    rows_pt = NP // _NS
    epcp = nch * _CH
    EP = epcp * _NS
    ngrp = nch // 8

    @functools.partial(
        pl.kernel,
        mesh=_sc_mesh(),
        out_type=jax.ShapeDtypeStruct((2 * NP, 16), jnp.float32),
        scratch_types=(
            [pltpu.VMEM((_CH, 16), jnp.float32)]
            + [pltpu.VMEM((_CH,), jnp.int32) for _ in range(16)]
            + [
                pltpu.VMEM_SHARED((NP, 16), jnp.float32),
                pltpu.SemaphoreType.DMA,
                pltpu.SemaphoreType.DMA,
            ]
        ),
    )
    def deg_kernel(dst_hbm, ones_hbm, init_hbm, out_hbm, *scr):
        ones_v = scr[0]
        didx = [scr[1 + i] for i in range(16)]
        acc_sh = scr[17]
        isem = scr[18]
        zsem = scr[19]
        cid = lax.axis_index("c")
        sid = lax.axis_index("s")
        r0 = sid * rows_pt
        ebase = cid * EP + sid * epcp
        zcopy = pltpu.async_copy(
            init_hbm.at[pl.ds(r0, rows_pt)], acc_sh.at[pl.ds(r0, rows_pt)],
            zsem,
        )
        pltpu.sync_copy(ones_hbm, ones_v)
        for j in range(8):
            pltpu.async_copy(
                dst_hbm.at[pl.ds(ebase + j * _CH, _CH)], didx[j], isem
            )
        for j in range(8):
            pltpu.make_async_copy(
                dst_hbm.at[pl.ds(ebase + j * _CH, _CH)], didx[j], isem
            ).wait()
        zcopy.wait()
        plsc.subcore_barrier()

        def grp_pair(gp, carry):
            for half in range(2):
                g = 2 * gp + half
                other = 1 - half
                for j in range(8):
                    if j == 4:
                        @pl.when(g < ngrp - 1)
                        def _(g=g, other=other):
                            for jj in range(8):
                                nc = (g + 1) * 8 + jj
                                pltpu.async_copy(
                                    dst_hbm.at[pl.ds(ebase + nc * _CH, _CH)],
                                    didx[other * 8 + jj], isem,
                                )
                    if j == 6:
                        @pl.when(g < ngrp - 1)
                        def _(g=g, other=other):
                            for jj in range(8):
                                nc = (g + 1) * 8 + jj
                                pltpu.make_async_copy(
                                    dst_hbm.at[pl.ds(ebase + nc * _CH, _CH)],
                                    didx[other * 8 + jj], isem,
                                ).wait()
                    pltpu.sync_copy(
                        ones_v, acc_sh.at[didx[half * 8 + j]], add=True
                    )
            return carry

        lax.fori_loop(0, ngrp // 2, grp_pair, 0)
        plsc.subcore_barrier()
        pltpu.sync_copy(
            acc_sh.at[pl.ds(r0, rows_pt)],
            out_hbm.at[pl.ds(cid * NP + r0, rows_pt)],
        )

    return deg_kernel


def _make_agg_kernel(NP, nch, D):
    rows_pt = NP // _NS
    epcp = nch * _CH
    EP = epcp * _NS
    ngrp = nch // 8

    @functools.partial(
        pl.kernel,
        mesh=_sc_mesh(),
        out_type=jax.ShapeDtypeStruct((2 * NP, D), jnp.float32),
        scratch_types=(
            [pltpu.VMEM((_CH, D), jnp.float32) for _ in range(2)]
            + [pltpu.VMEM((_CH,), jnp.int32) for _ in range(32)]
            + [pltpu.VMEM_SHARED((NP, D), jnp.float32)]
            + [pltpu.SemaphoreType.DMA for _ in range(4)]
        ),
    )
    def agg_kernel(tab_hbm, src_hbm, dst_hbm, zeros_hbm, out_hbm, *scr):
        rows = [scr[0], scr[1]]
        sidx = [scr[2 + i] for i in range(16)]
        didx = [scr[18 + i] for i in range(16)]
        acc_sh = scr[34]
        gsem = [scr[35], scr[36]]
        isem = scr[37]
        zsem = scr[38]
        cid = lax.axis_index("c")
        sid = lax.axis_index("s")
        rr = sid * rows_pt
        ebase = cid * EP + sid * epcp
        zcopy = pltpu.async_copy(
            zeros_hbm.at[pl.ds(rr, rows_pt)], acc_sh.at[pl.ds(rr, rows_pt)],
            zsem,
        )
        for j in range(8):
            pltpu.async_copy(
                src_hbm.at[pl.ds(ebase + j * _CH, _CH)], sidx[j], isem
            )
            pltpu.async_copy(
                dst_hbm.at[pl.ds(ebase + j * _CH, _CH)], didx[j], isem
            )
        for j in range(8):
            pltpu.make_async_copy(
                src_hbm.at[pl.ds(ebase + j * _CH, _CH)], sidx[j], isem
            ).wait()
            pltpu.make_async_copy(
                dst_hbm.at[pl.ds(ebase + j * _CH, _CH)], didx[j], isem
            ).wait()
        pltpu.async_copy(tab_hbm.at[sidx[0]], rows[0], gsem[0])
        zcopy.wait()
        plsc.subcore_barrier()

        def grp_pair(gp, carry):
            handles = {}
            for half in range(2):
                g = 2 * gp + half
                other = 1 - half
                for j in range(8):
                    b = j % 2
                    nb = 1 - b
                    if (half, j) == (0, 0):
                        pltpu.make_async_copy(
                            tab_hbm.at[sidx[0]], rows[0], gsem[0]
                        ).wait()
                    else:
                        handles.pop(b).wait()
                    if j < 7:
                        handles[nb] = pltpu.async_copy(
                            tab_hbm.at[sidx[half * 8 + j + 1]], rows[nb],
                            gsem[nb],
                        )
                    elif half == 0:
                        handles[nb] = pltpu.async_copy(
                            tab_hbm.at[sidx[8]], rows[nb], gsem[nb]
                        )
                    else:
                        @pl.when(g < ngrp - 1)
                        def _(nb=nb):
                            pltpu.async_copy(
                                tab_hbm.at[sidx[0]], rows[nb], gsem[nb]
                            )
                    if j == 4:
                        @pl.when(g < ngrp - 1)
                        def _(g=g, other=other):
                            for jj in range(8):
                                nc = (g + 1) * 8 + jj
                                pltpu.async_copy(
                                    src_hbm.at[pl.ds(ebase + nc * _CH, _CH)],
                                    sidx[other * 8 + jj], isem,
                                )
                                pltpu.async_copy(
                                    dst_hbm.at[pl.ds(ebase + nc * _CH, _CH)],
                                    didx[other * 8 + jj], isem,
                                )
                    if j == 6:
                        @pl.when(g < ngrp - 1)
                        def _(g=g, other=other):
                            for jj in range(8):
                                nc = (g + 1) * 8 + jj
                                pltpu.make_async_copy(
                                    src_hbm.at[pl.ds(ebase + nc * _CH, _CH)],
                                    sidx[other * 8 + jj], isem,
                                ).wait()
                                pltpu.make_async_copy(
                                    dst_hbm.at[pl.ds(ebase + nc * _CH, _CH)],
                                    didx[other * 8 + jj], isem,
                                ).wait()
                    pltpu.sync_copy(
                        rows[b], acc_sh.at[didx[half * 8 + j]], add=True
                    )
            return carry

        lax.fori_loop(0, ngrp // 2, grp_pair, 0)
        plsc.subcore_barrier()
        pltpu.sync_copy(
            acc_sh.at[pl.ds(rr, rows_pt)],
            out_hbm.at[pl.ds(cid * NP + rr, rows_pt)],
        )

    return agg_kernel


def _make_state_gather_kernel(B, D):
    bpw = B // (_NC * _NS)

    @functools.partial(
        pl.kernel,
        mesh=_sc_mesh(),
        out_type=jax.ShapeDtypeStruct((B, D), jnp.float32),
        scratch_types=[
            pltpu.VMEM((bpw,), jnp.int32),
            pltpu.VMEM((bpw, D), jnp.float32),
            pltpu.SemaphoreType.DMA,
        ],
    )
    def gather_kernel(tab_hbm, idx_hbm, out_hbm, idx_v, rows_v, sem):
        wid = lax.axis_index("s") * _NC + lax.axis_index("c")
        base = wid * bpw
        pltpu.sync_copy(idx_hbm.at[pl.ds(base, bpw)], idx_v)
        pltpu.async_copy(tab_hbm.at[idx_v], rows_v, sem).wait()
        pltpu.sync_copy(rows_v, out_hbm.at[pl.ds(base, bpw)])

    return gather_kernel


def _prep_body(deg_ref, x_ref, dinv_ref, y0_ref):
    dinv = lax.rsqrt(deg_ref[:, 0:1])
    dinv_ref[:, :] = dinv
    y0_ref[:, :] = x_ref[:, :] * dinv


def _mid_body(agg_ref, y0_ref, dinv_ref, w1_ref, b1_ref, w2_ref, u_ref):
    dinv = dinv_ref[:, :]
    ax = (agg_ref[:, :] + y0_ref[:, :]) * dinv
    h1 = jnp.maximum(
        jnp.dot(ax, w1_ref[:, :], preferred_element_type=jnp.float32) + b1_ref[:],
        0.0,
    )
    u_ref[:, :] = (
        jnp.dot(h1, w2_ref[:, :], preferred_element_type=jnp.float32)
        * dinv
    )


def _final_body(agg_ref, u_ref, dinv_ref, b2_ref, g_ref):
    dinv = dinv_ref[:, :]
    g_ref[:, :] = dinv * (agg_ref[:, :] + u_ref[:, :]) + b2_ref[:]


def _head_body(gx_ref, gy_ref, wh_ref, bh_ref, wp_ref, bp_ref, out_ref):
    z = gx_ref[:, :] * gy_ref[:, :]
    o = jnp.maximum(
        jnp.dot(z, wh_ref[:, :], preferred_element_type=jnp.float32) + bh_ref[:],
        0.0,
    )
    p = jnp.dot(o, wp_ref[:, :], preferred_element_type=jnp.float32) + bp_ref[:]
    m = jnp.max(p, axis=1, keepdims=True)
    e = jnp.exp(p - m)
    out_ref[:, :] = e / jnp.sum(e, axis=1, keepdims=True)


def kernel(x_a, x_b, W1, b1, W2, b2, Wh, bh, Wp, bp,
           edge_index_a, edge_index_b, states):
    N, D = x_a.shape
    E = edge_index_a.shape[1]
    S = states.shape[0]
    H = W1.shape[1]
    NP = ((N + 511) // 512) * 512
    TN = 2 * NP

    epc = E // _NS
    nch = ((-(-epc // _CH) + 15) // 16) * 16
    EP = nch * _CH * _NS
    padn = EP - E

    X = jnp.zeros((TN, D), jnp.float32)
    X = lax.dynamic_update_slice(X, x_a, (0, 0))
    X = lax.dynamic_update_slice(X, x_b, (NP, 0))
    fill_src = jnp.zeros((padn,), jnp.int32)
    fill_dst = N + (jnp.arange(padn, dtype=jnp.int32) % (NP - N))
    src_all = jnp.concatenate([edge_index_a[0], fill_src,
                               edge_index_b[0] + NP, fill_src])
    dst_all = jnp.concatenate([edge_index_a[1], fill_dst,
                               edge_index_b[1], fill_dst])
    zerosD = jnp.zeros((NP, D), jnp.float32)
    ones16 = jnp.ones((_CH, 16), jnp.float32)
    ones_init = jnp.ones((NP, 16), jnp.float32)
    sidx = jnp.concatenate([states[:, 0], NP + states[:, 1]])

    deg16 = _make_deg_kernel(NP, nch)(dst_all, ones16, ones_init)

    R = 1024
    grid = (TN // R,)
    dinv, Y0 = pl.pallas_call(
        _prep_body,
        grid=grid,
        in_specs=[
            pl.BlockSpec((R, 16), lambda i: (i, 0)),
            pl.BlockSpec((R, D), lambda i: (i, 0)),
        ],
        out_specs=[
            pl.BlockSpec((R, 1), lambda i: (i, 0)),
            pl.BlockSpec((R, D), lambda i: (i, 0)),
        ],
        out_shape=[
            jax.ShapeDtypeStruct((TN, 1), jnp.float32),
            jax.ShapeDtypeStruct((TN, D), jnp.float32),
        ],
    )(deg16, X)

    agg_kernel = _make_agg_kernel(NP, nch, D)

    agg1 = agg_kernel(Y0, src_all, dst_all, zerosD)

    U = pl.pallas_call(
        _mid_body,
        grid=grid,
        in_specs=[
            pl.BlockSpec((R, D), lambda i: (i, 0)),
            pl.BlockSpec((R, D), lambda i: (i, 0)),
            pl.BlockSpec((R, 1), lambda i: (i, 0)),
            pl.BlockSpec((D, H), lambda i: (0, 0)),
            pl.BlockSpec((H,), lambda i: (0,)),
            pl.BlockSpec((H, D), lambda i: (0, 0)),
        ],
        out_specs=pl.BlockSpec((R, D), lambda i: (i, 0)),
        out_shape=jax.ShapeDtypeStruct((TN, D), jnp.float32),
    )(agg1, Y0, dinv, W1, b1, W2)

    agg2 = agg_kernel(U, src_all, dst_all, zerosD)

    G = pl.pallas_call(
        _final_body,
        grid=grid,
        in_specs=[
            pl.BlockSpec((R, D), lambda i: (i, 0)),
            pl.BlockSpec((R, D), lambda i: (i, 0)),
            pl.BlockSpec((R, 1), lambda i: (i, 0)),
            pl.BlockSpec((D,), lambda i: (0,)),
        ],
        out_specs=pl.BlockSpec((R, D), lambda i: (i, 0)),
        out_shape=jax.ShapeDtypeStruct((TN, D), jnp.float32),
    )(agg2, U, dinv, b2)

    gxy = _make_state_gather_kernel(2 * S, D)(G, sidx)

    policy = pl.pallas_call(
        _head_body,
        out_shape=jax.ShapeDtypeStruct((S, 2), jnp.float32),
    )(gxy[:S], gxy[S:], Wh, bh, Wp, bp)
    return policy


# speedup vs baseline: 1.3133x; 1.2658x over previous
import functools

import jax
import jax.numpy as jnp
from jax import lax
from jax.experimental import pallas as pl
from jax.experimental.pallas import tpu as pltpu
from jax.experimental.pallas import tpu_sc as plsc

_NC = 2
_NS = 16
_CH = 128


def _sc_mesh():
    return plsc.VectorSubcoreMesh(
        core_axis_name="c", subcore_axis_name="s", num_cores=_NC, num_subcores=_NS
    )


def _wait_gather(tab_hbm, rows_ref, sem):
    pltpu.make_async_copy(tab_hbm.at[pl.ds(0, _CH)], rows_ref, sem).wait()


def _wait_scatter(rows_ref, acc_sh, sem):
    pltpu.make_async_copy(rows_ref, acc_sh.at[pl.ds(0, _CH)], sem).wait()


def _make_deg_kernel(NP, nch):
    rows_pt = NP // _NS
    epcp = nch * _CH
    EP = epcp * _NS
    ngrp = nch // 8

    @functools.partial(
        pl.kernel,
        mesh=_sc_mesh(),
        out_type=jax.ShapeDtypeStruct((2 * NP, 16), jnp.float32),
        scratch_types=(
            [pltpu.VMEM((_CH, 16), jnp.float32)]
            + [pltpu.VMEM((_CH,), jnp.int32) for _ in range(16)]
            + [
                pltpu.VMEM_SHARED((NP, 16), jnp.float32),
                pltpu.SemaphoreType.DMA,
                pltpu.SemaphoreType.DMA,
            ]
        ),
    )
    def deg_kernel(dst_hbm, ones_hbm, init_hbm, out_hbm, *scr):
        ones_v = scr[0]
        didx = [scr[1 + i] for i in range(16)]
        acc_sh = scr[17]
        isem = scr[18]
        zsem = scr[19]
        cid = lax.axis_index("c")
        sid = lax.axis_index("s")
        r0 = sid * rows_pt
        ebase = cid * EP + sid * epcp
        zcopy = pltpu.async_copy(
            init_hbm.at[pl.ds(r0, rows_pt)], acc_sh.at[pl.ds(r0, rows_pt)],
            zsem,
        )
        pltpu.sync_copy(ones_hbm, ones_v)
        for j in range(8):
            pltpu.async_copy(
                dst_hbm.at[pl.ds(ebase + j * _CH, _CH)], didx[j], isem
            )
        for j in range(8):
            pltpu.make_async_copy(
                dst_hbm.at[pl.ds(ebase + j * _CH, _CH)], didx[j], isem
            ).wait()
        zcopy.wait()
        plsc.subcore_barrier()

        def grp_pair(gp, carry):
            for half in range(2):
                g = 2 * gp + half
                other = 1 - half
                for j in range(8):
                    if j == 4:
                        @pl.when(g < ngrp - 1)
                        def _(g=g, other=other):
                            for jj in range(8):
                                nc = (g + 1) * 8 + jj
                                pltpu.async_copy(
                                    dst_hbm.at[pl.ds(ebase + nc * _CH, _CH)],
                                    didx[other * 8 + jj], isem,
                                )
                    if j == 6:
                        @pl.when(g < ngrp - 1)
                        def _(g=g, other=other):
                            for jj in range(8):
                                nc = (g + 1) * 8 + jj
                                pltpu.make_async_copy(
                                    dst_hbm.at[pl.ds(ebase + nc * _CH, _CH)],
                                    didx[other * 8 + jj], isem,
                                ).wait()
                    pltpu.sync_copy(
                        ones_v, acc_sh.at[didx[half * 8 + j]], add=True
                    )
            return carry

        lax.fori_loop(0, ngrp // 2, grp_pair, 0)
        plsc.subcore_barrier()
        pltpu.sync_copy(
            acc_sh.at[pl.ds(r0, rows_pt)],
            out_hbm.at[pl.ds(cid * NP + r0, rows_pt)],
        )

    return deg_kernel


def _make_agg_kernel(NP, E, D):
    CH = 80
    rows_pt = NP // _NS
    epc = E // _NS
    n_chunks = epc // CH

    @functools.partial(
        pl.kernel,
        mesh=_sc_mesh(),
        out_type=jax.ShapeDtypeStruct((2 * NP, D), jnp.float32),
        scratch_types=[
            pltpu.VMEM((CH,), jnp.int32),
            pltpu.VMEM((CH,), jnp.int32),
            pltpu.VMEM((CH, D), jnp.float32),
            pltpu.VMEM_SHARED((NP, D), jnp.float32),
            pltpu.SemaphoreType.DMA,
        ],
    )
    def agg_kernel(tab_hbm, src_hbm, dst_hbm, zeros_hbm, out_hbm,
                   si_v, di_v, rows_v, acc_sh, sem):
        cid = lax.axis_index("c")
        sid = lax.axis_index("s")
        rr = sid * rows_pt
        ebase = cid * (src_hbm.shape[0] // 2) + sid * epc
        pltpu.sync_copy(
            zeros_hbm.at[pl.ds(rr, rows_pt)], acc_sh.at[pl.ds(rr, rows_pt)]
        )
        plsc.subcore_barrier()

        def body(i, carry):
            base = ebase + i * CH
            pltpu.sync_copy(src_hbm.at[pl.ds(base, CH)], si_v)
            pltpu.sync_copy(dst_hbm.at[pl.ds(base, CH)], di_v)
            pltpu.async_copy(tab_hbm.at[si_v], rows_v, sem).wait()
            pltpu.sync_copy(rows_v, acc_sh.at[di_v], add=True)
            return carry

        lax.fori_loop(0, n_chunks, body, 0)
        plsc.subcore_barrier()
        pltpu.sync_copy(
            acc_sh.at[pl.ds(rr, rows_pt)],
            out_hbm.at[pl.ds(cid * NP + rr, rows_pt)],
        )

    return agg_kernel


def _make_state_gather_kernel(B, D):
    bpw = B // (_NC * _NS)

    @functools.partial(
        pl.kernel,
        mesh=_sc_mesh(),
        out_type=jax.ShapeDtypeStruct((B, D), jnp.float32),
        scratch_types=[
            pltpu.VMEM((bpw,), jnp.int32),
            pltpu.VMEM((bpw, D), jnp.float32),
            pltpu.SemaphoreType.DMA,
        ],
    )
    def gather_kernel(tab_hbm, idx_hbm, out_hbm, idx_v, rows_v, sem):
        wid = lax.axis_index("s") * _NC + lax.axis_index("c")
        base = wid * bpw
        pltpu.sync_copy(idx_hbm.at[pl.ds(base, bpw)], idx_v)
        pltpu.async_copy(tab_hbm.at[idx_v], rows_v, sem).wait()
        pltpu.sync_copy(rows_v, out_hbm.at[pl.ds(base, bpw)])

    return gather_kernel


def _prep_body(deg_ref, x_ref, dinv_ref, y0_ref):
    dinv = lax.rsqrt(deg_ref[:, 0:1])
    dinv_ref[:, :] = dinv
    y0_ref[:, :] = x_ref[:, :] * dinv


def _mid_body(agg_ref, y0_ref, dinv_ref, w1_ref, b1_ref, w2_ref, u_ref):
    dinv = dinv_ref[:, :]
    ax = (agg_ref[:, :] + y0_ref[:, :]) * dinv
    h1 = jnp.maximum(
        jnp.dot(ax, w1_ref[:, :], preferred_element_type=jnp.float32) + b1_ref[:],
        0.0,
    )
    u_ref[:, :] = (
        jnp.dot(h1, w2_ref[:, :], preferred_element_type=jnp.float32)
        * dinv
    )


def _final_body(agg_ref, u_ref, dinv_ref, b2_ref, g_ref):
    dinv = dinv_ref[:, :]
    g_ref[:, :] = dinv * (agg_ref[:, :] + u_ref[:, :]) + b2_ref[:]


def _head_body(gx_ref, gy_ref, wh_ref, bh_ref, wp_ref, bp_ref, out_ref):
    z = gx_ref[:, :] * gy_ref[:, :]
    o = jnp.maximum(
        jnp.dot(z, wh_ref[:, :], preferred_element_type=jnp.float32) + bh_ref[:],
        0.0,
    )
    p = jnp.dot(o, wp_ref[:, :], preferred_element_type=jnp.float32) + bp_ref[:]
    m = jnp.max(p, axis=1, keepdims=True)
    e = jnp.exp(p - m)
    out_ref[:, :] = e / jnp.sum(e, axis=1, keepdims=True)


def kernel(x_a, x_b, W1, b1, W2, b2, Wh, bh, Wp, bp,
           edge_index_a, edge_index_b, states):
    N, D = x_a.shape
    E = edge_index_a.shape[1]
    S = states.shape[0]
    H = W1.shape[1]
    NP = ((N + 511) // 512) * 512
    TN = 2 * NP

    epc = E // _NS
    nch = ((-(-epc // _CH) + 15) // 16) * 16
    EP = nch * _CH * _NS
    padn = EP - E

    X = jnp.zeros((TN, D), jnp.float32)
    X = lax.dynamic_update_slice(X, x_a, (0, 0))
    X = lax.dynamic_update_slice(X, x_b, (NP, 0))
    fill_src = jnp.zeros((padn,), jnp.int32)
    fill_dst = N + (jnp.arange(padn, dtype=jnp.int32) % (NP - N))
    src_all = jnp.concatenate([edge_index_a[0], fill_src,
                               edge_index_b[0] + NP, fill_src])
    dst_all = jnp.concatenate([edge_index_a[1], fill_dst,
                               edge_index_b[1], fill_dst])
    zerosD = jnp.zeros((NP, D), jnp.float32)
    ones16 = jnp.ones((_CH, 16), jnp.float32)
    ones_init = jnp.ones((NP, 16), jnp.float32)
    sidx = jnp.concatenate([states[:, 0], NP + states[:, 1]])

    deg16 = _make_deg_kernel(NP, nch)(dst_all, ones16, ones_init)

    R = 1024
    grid = (TN // R,)
    dinv, Y0 = pl.pallas_call(
        _prep_body,
        grid=grid,
        in_specs=[
            pl.BlockSpec((R, 16), lambda i: (i, 0)),
            pl.BlockSpec((R, D), lambda i: (i, 0)),
        ],
        out_specs=[
            pl.BlockSpec((R, 1), lambda i: (i, 0)),
            pl.BlockSpec((R, D), lambda i: (i, 0)),
        ],
        out_shape=[
            jax.ShapeDtypeStruct((TN, 1), jnp.float32),
            jax.ShapeDtypeStruct((TN, D), jnp.float32),
        ],
    )(deg16, X)

    agg_kernel = _make_agg_kernel(NP, E, D)

    agg1 = agg_kernel(Y0, src_all, dst_all, zerosD)

    U = pl.pallas_call(
        _mid_body,
        grid=grid,
        in_specs=[
            pl.BlockSpec((R, D), lambda i: (i, 0)),
            pl.BlockSpec((R, D), lambda i: (i, 0)),
            pl.BlockSpec((R, 1), lambda i: (i, 0)),
            pl.BlockSpec((D, H), lambda i: (0, 0)),
            pl.BlockSpec((H,), lambda i: (0,)),
            pl.BlockSpec((H, D), lambda i: (0, 0)),
        ],
        out_specs=pl.BlockSpec((R, D), lambda i: (i, 0)),
        out_shape=jax.ShapeDtypeStruct((TN, D), jnp.float32),
    )(agg1, Y0, dinv, W1, b1, W2)

    agg2 = agg_kernel(U, src_all, dst_all, zerosD)

    G = pl.pallas_call(
        _final_body,
        grid=grid,
        in_specs=[
            pl.BlockSpec((R, D), lambda i: (i, 0)),
            pl.BlockSpec((R, D), lambda i: (i, 0)),
            pl.BlockSpec((R, 1), lambda i: (i, 0)),
            pl.BlockSpec((D,), lambda i: (0,)),
        ],
        out_specs=pl.BlockSpec((R, D), lambda i: (i, 0)),
        out_shape=jax.ShapeDtypeStruct((TN, D), jnp.float32),
    )(agg2, U, dinv, b2)

    gxy = _make_state_gather_kernel(2 * S, D)(G, sidx)

    policy = pl.pallas_call(
        _head_body,
        out_shape=jax.ShapeDtypeStruct((S, 2), jnp.float32),
    )(gxy[:S], gxy[S:], Wh, bh, Wp, bp)
    return policy
